# baseline (speedup 1.0000x reference)
import functools

import jax
import jax.numpy as jnp
from jax import lax
from jax.experimental import pallas as pl
from jax.experimental.pallas import tpu as pltpu
from jax.experimental.pallas import tpu_sc as plsc

_N = 10000
_D = 256
_E = 160000


def _minmax_body(x_ref, mn_ref, mx_ref):
    i = pl.program_id(0)
    bmn = jnp.min(x_ref[...])
    bmx = jnp.max(x_ref[...])

    @pl.when(i == 0)
    def _():
        mn_ref[0, 0] = bmn
        mx_ref[0, 0] = bmx

    @pl.when(i > 0)
    def _():
        mn_ref[0, 0] = jnp.minimum(mn_ref[0, 0], bmn)
        mx_ref[0, 0] = jnp.maximum(mx_ref[0, 0], bmx)


def _minmax2_body(a_ref, b_ref, mna_ref, mxa_ref, mnb_ref, mxb_ref):
    i = pl.program_id(0)
    bmna = jnp.min(a_ref[...])
    bmxa = jnp.max(a_ref[...])
    bmnb = jnp.min(b_ref[...])
    bmxb = jnp.max(b_ref[...])

    @pl.when(i == 0)
    def _():
        mna_ref[0, 0] = bmna
        mxa_ref[0, 0] = bmxa
        mnb_ref[0, 0] = bmnb
        mxb_ref[0, 0] = bmxb

    @pl.when(i > 0)
    def _():
        mna_ref[0, 0] = jnp.minimum(mna_ref[0, 0], bmna)
        mxa_ref[0, 0] = jnp.maximum(mxa_ref[0, 0], bmxa)
        mnb_ref[0, 0] = jnp.minimum(mnb_ref[0, 0], bmnb)
        mxb_ref[0, 0] = jnp.maximum(mxb_ref[0, 0], bmxb)


def _minmax2(a, b):
    m, c = a.shape
    bm = 2000
    outs = pl.pallas_call(
        _minmax2_body,
        grid=(m // bm,),
        in_specs=[
            pl.BlockSpec((bm, c), lambda i: (i, 0)),
            pl.BlockSpec((bm, c), lambda i: (i, 0)),
        ],
        out_specs=[pl.BlockSpec(memory_space=pltpu.SMEM)] * 4,
        out_shape=[jax.ShapeDtypeStruct((1, 1), jnp.float32)] * 4,
    )(a, b)
    return tuple(o[0, 0] for o in outs)


def _minmax(x):
    m, c = x.shape
    bm = 2000
    mn, mx = pl.pallas_call(
        _minmax_body,
        grid=(m // bm,),
        in_specs=[pl.BlockSpec((bm, c), lambda i: (i, 0))],
        out_specs=[
            pl.BlockSpec(memory_space=pltpu.SMEM),
            pl.BlockSpec(memory_space=pltpu.SMEM),
        ],
        out_shape=[jax.ShapeDtypeStruct((1, 1), jnp.float32)] * 2,
    )(x)
    return mn[0, 0], mx[0, 0]


def _affine_mm_body(sb_ref, x_ref, w_ref, o_ref):
    xb = x_ref[...] * sb_ref[0] + sb_ref[1]
    o_ref[...] = jnp.dot(xb, w_ref[...], preferred_element_type=jnp.float32)


def _relu_mm_body(x_ref, w_ref, o_ref):
    xb = jnp.maximum(x_ref[...], 0.0)
    o_ref[...] = jnp.dot(xb, w_ref[...], preferred_element_type=jnp.float32)


def _affine_matmul_split(x, w, s, b):
    m, k = x.shape
    n = w.shape[1]
    bm = 2000
    nj = n // 128
    sb = jnp.stack([s, b]).astype(jnp.float32)
    return pl.pallas_call(
        _affine_mm_body,
        grid=(nj, m // bm),
        in_specs=[
            pl.BlockSpec(memory_space=pltpu.SMEM),
            pl.BlockSpec((bm, k), lambda j, i: (i, 0)),
            pl.BlockSpec((k, 128), lambda j, i: (0, j)),
        ],
        out_specs=pl.BlockSpec((bm, 128), lambda j, i: (j * (m // bm) + i, 0)),
        out_shape=jax.ShapeDtypeStruct((nj * m, 128), jnp.float32),
    )(sb, x, w)


def _relu_matmul_split(x, w):
    m, k = x.shape
    n = w.shape[1]
    bm = 2000
    nj = n // 128
    return pl.pallas_call(
        _relu_mm_body,
        grid=(nj, m // bm),
        in_specs=[
            pl.BlockSpec((bm, k), lambda j, i: (i, 0)),
            pl.BlockSpec((k, 128), lambda j, i: (0, j)),
        ],
        out_specs=pl.BlockSpec((bm, 128), lambda j, i: (j * (m // bm) + i, 0)),
        out_shape=jax.ShapeDtypeStruct((nj * m, 128), jnp.float32),
    )(x, w)


_CHUNK = 88
_NCH = 114
_EPT = _NCH * _CHUNK
_NP = 10112
_DRN = 632
_DRNL = 520


def _spmm_sc_kernel(y_hbm, rowr_hbm, colr_hbm, out_hbm,
                    ridx_v, cidx_v, rows0_v, rows1_v, acc_sh, gsem, ssem):
    c = lax.axis_index("c")
    s = lax.axis_index("s")

    pltpu.sync_copy(rowr_hbm.at[s], ridx_v)
    pltpu.sync_copy(colr_hbm.at[c * 16 + s], cidx_v)

    def zfill(i, _):
        for j in range(8):
            rows0_v[i, pl.ds(j * 16, 16)] = jnp.zeros((16,), jnp.float32)
        return 0

    lax.fori_loop(0, _CHUNK, zfill, 0)
    r0 = s * _DRN
    nfull = _DRN // _CHUNK
    for i in range(nfull):
        pltpu.sync_copy(rows0_v, acc_sh.at[pl.ds(r0 + i * _CHUNK, _CHUNK)])
    rem = _DRN - nfull * _CHUNK
    pltpu.sync_copy(rows0_v.at[pl.ds(0, rem)],
                    acc_sh.at[pl.ds(r0 + nfull * _CHUNK, rem)])
    plsc.subcore_barrier()

    def gfire(k, buf):
        pltpu.async_copy(y_hbm.at[cidx_v.at[pl.ds(k * _CHUNK, _CHUNK)]], buf, gsem)

    def gwait(k, buf):
        pltpu.make_async_copy(
            y_hbm.at[cidx_v.at[pl.ds(k * _CHUNK, _CHUNK)]], buf, gsem).wait()

    def sfire(k, buf):
        pltpu.async_copy(buf, acc_sh.at[ridx_v.at[k]], ssem, add=True)

    def swait(k, buf):
        pltpu.make_async_copy(buf, acc_sh.at[ridx_v.at[k]], ssem).wait()

    gfire(0, rows0_v)
    gfire(1, rows1_v)

    def body(j, _):
        k0 = 2 * j
        k1 = k0 + 1
        gwait(k0, rows0_v)
        sfire(k0, rows0_v)
        gwait(k1, rows1_v)
        swait(k0, rows0_v)
        gfire(k0 + 2, rows0_v)
        sfire(k1, rows1_v)
        swait(k1, rows1_v)
        gfire(k1 + 2, rows1_v)
        return 0

    lax.fori_loop(0, _NCH // 2 - 1, body, 0)
    k0 = _NCH - 2
    k1 = _NCH - 1
    gwait(k0, rows0_v)
    sfire(k0, rows0_v)
    gwait(k1, rows1_v)
    swait(k0, rows0_v)
    sfire(k1, rows1_v)
    swait(k1, rows1_v)

    plsc.subcore_barrier()

    @pl.when(s < 15)
    def _():
        pltpu.sync_copy(acc_sh.at[pl.ds(r0, _DRN)],
                        out_hbm.at[pl.ds(r0, _DRN), pl.ds(c * 128, 128)])

    @pl.when(s == 15)
    def _():
        pltpu.sync_copy(acc_sh.at[pl.ds(r0, _DRNL)],
                        out_hbm.at[pl.ds(r0, _DRNL), pl.ds(c * 128, 128)])


def _spmm(ysplit, row, col, base):
    n = _N
    pad = _EPT - (_E // 16)
    row2 = jnp.pad(row.reshape(16, _E // 16), ((0, 0), (0, pad)),
                   constant_values=n)
    col2 = jnp.pad(col.reshape(16, _E // 16), ((0, 0), (0, pad)),
                   constant_values=0) + base
    rowr = row2.reshape(16, _NCH, _CHUNK)
    colcat = jnp.concatenate([col2, col2 + n]).reshape(32, _NCH * _CHUNK)
    ch = 128
    mesh = plsc.VectorSubcoreMesh(core_axis_name="c", subcore_axis_name="s")
    run = pl.kernel(
        _spmm_sc_kernel,
        out_type=jax.ShapeDtypeStruct((n, 256), jnp.float32),
        mesh=mesh,
        scratch_types=[
            pltpu.VMEM((_NCH, _CHUNK), jnp.int32),
            pltpu.VMEM((_NCH * _CHUNK,), jnp.int32),
            pltpu.VMEM((_CHUNK, ch), jnp.float32),
            pltpu.VMEM((_CHUNK, ch), jnp.float32),
            pltpu.VMEM_SHARED((_NP, ch), jnp.float32),
            pltpu.SemaphoreType.DMA,
            pltpu.SemaphoreType.DMA,
        ],
    )
    return run(ysplit, rowr, colcat)


def _kld_body(sc_ref, mu_ref, lv_ref, o_ref):
    i = pl.program_id(0)
    mun = mu_ref[...] * sc_ref[0] + sc_ref[1]
    lvn = lv_ref[...] * sc_ref[2] + sc_ref[3]
    e = jnp.exp(lvn)
    t = 1.0 + 2.0 * lvn - mun * mun - e * e
    part = jnp.sum(t)

    @pl.when(i == 0)
    def _():
        o_ref[0, 0] = part

    @pl.when(i > 0)
    def _():
        o_ref[0, 0] = o_ref[0, 0] + part


def _kld_sum(mu, lv, s_mu, b_mu, s_lv, b_lv):
    m, c = mu.shape
    bm = 2000
    sc = jnp.stack([s_mu, b_mu, s_lv, b_lv]).astype(jnp.float32)
    tot = pl.pallas_call(
        _kld_body,
        grid=(m // bm,),
        in_specs=[
            pl.BlockSpec(memory_space=pltpu.SMEM),
            pl.BlockSpec((bm, c), lambda i: (i, 0)),
            pl.BlockSpec((bm, c), lambda i: (i, 0)),
        ],
        out_specs=pl.BlockSpec(memory_space=pltpu.SMEM),
        out_shape=jax.ShapeDtypeStruct((1, 1), jnp.float32),
    )(sc, mu, lv)
    return tot[0, 0]


def _clean(z):
    z = jnp.where(z == -jnp.inf, 0.0, z)
    return jnp.where(jnp.isnan(z), 0.0, z)


def _zzt_body(z1_ref, z2_ref, o_ref):
    a = _clean(z1_ref[...])
    b = _clean(z2_ref[...])
    o_ref[...] = lax.dot_general(
        a, b, (((1,), (1,)), ((), ())), preferred_element_type=jnp.float32
    )


def _zzt(z):
    m, k = z.shape
    bm = 2000
    bn = 2048
    return pl.pallas_call(
        _zzt_body,
        grid=(m // bm, pl.cdiv(m, bn)),
        in_specs=[
            pl.BlockSpec((bm, k), lambda i, j: (i, 0)),
            pl.BlockSpec((bn, k), lambda i, j: (j, 0)),
        ],
        out_specs=pl.BlockSpec((bm, bn), lambda i, j: (i, j)),
        out_shape=jax.ShapeDtypeStruct((m, m), jnp.float32),
    )(z, z)


def kernel(feature, edge_index, W1, W2, W3):
    n = feature.shape[0]
    row = edge_index[0]
    col = edge_index[1]

    mn, mx = _minmax(feature)
    s = 2.0 / (mx - mn)
    b = -2.0 * mn / (mx - mn) - 1.0

    y1s = _affine_matmul_split(feature, W1, s, b)
    s1 = _spmm(y1s, row, col, 0)
    y23s = _relu_matmul_split(s1, jnp.concatenate([W2, W3], axis=1))
    mu = _spmm(y23s, row, col, 0)
    logvar = _spmm(y23s, row, col, 2 * _N)

    adj_recon = _zzt(mu)

    mn_mu, mx_mu, mn_lv, mx_lv = _minmax2(mu, logvar)
    s_mu = 2.0 / (mx_mu - mn_mu)
    b_mu = -2.0 * mn_mu / (mx_mu - mn_mu) - 1.0
    s_lv = 2.0 / (mx_lv - mn_lv)
    b_lv = -2.0 * mn_lv / (mx_lv - mn_lv) - 1.0
    tot = _kld_sum(mu, logvar, s_mu, b_mu, s_lv, b_lv)
    kld = (-0.5 / n) * (tot / n)

    return (adj_recon, mu, logvar, mu, kld)


# speedup vs baseline: 1.0247x; 1.0247x over previous
import functools

import jax
import jax.numpy as jnp
from jax import lax
from jax.experimental import pallas as pl
from jax.experimental.pallas import tpu as pltpu
from jax.experimental.pallas import tpu_sc as plsc

_N = 10000
_D = 256
_E = 160000


def _minmax_body(x_ref, mn_ref, mx_ref):
    i = pl.program_id(0)
    bmn = jnp.min(x_ref[...])
    bmx = jnp.max(x_ref[...])

    @pl.when(i == 0)
    def _():
        mn_ref[0, 0] = bmn
        mx_ref[0, 0] = bmx

    @pl.when(i > 0)
    def _():
        mn_ref[0, 0] = jnp.minimum(mn_ref[0, 0], bmn)
        mx_ref[0, 0] = jnp.maximum(mx_ref[0, 0], bmx)


def _minmax2_body(a_ref, b_ref, mna_ref, mxa_ref, mnb_ref, mxb_ref):
    i = pl.program_id(0)
    bmna = jnp.min(a_ref[...])
    bmxa = jnp.max(a_ref[...])
    bmnb = jnp.min(b_ref[...])
    bmxb = jnp.max(b_ref[...])

    @pl.when(i == 0)
    def _():
        mna_ref[0, 0] = bmna
        mxa_ref[0, 0] = bmxa
        mnb_ref[0, 0] = bmnb
        mxb_ref[0, 0] = bmxb

    @pl.when(i > 0)
    def _():
        mna_ref[0, 0] = jnp.minimum(mna_ref[0, 0], bmna)
        mxa_ref[0, 0] = jnp.maximum(mxa_ref[0, 0], bmxa)
        mnb_ref[0, 0] = jnp.minimum(mnb_ref[0, 0], bmnb)
        mxb_ref[0, 0] = jnp.maximum(mxb_ref[0, 0], bmxb)


def _minmax2(a, b):
    m, c = a.shape
    bm = 2000
    outs = pl.pallas_call(
        _minmax2_body,
        grid=(m // bm,),
        in_specs=[
            pl.BlockSpec((bm, c), lambda i: (i, 0)),
            pl.BlockSpec((bm, c), lambda i: (i, 0)),
        ],
        out_specs=[pl.BlockSpec(memory_space=pltpu.SMEM)] * 4,
        out_shape=[jax.ShapeDtypeStruct((1, 1), jnp.float32)] * 4,
    )(a, b)
    return tuple(o[0, 0] for o in outs)


def _minmax(x):
    m, c = x.shape
    bm = 2000
    mn, mx = pl.pallas_call(
        _minmax_body,
        grid=(m // bm,),
        in_specs=[pl.BlockSpec((bm, c), lambda i: (i, 0))],
        out_specs=[
            pl.BlockSpec(memory_space=pltpu.SMEM),
            pl.BlockSpec(memory_space=pltpu.SMEM),
        ],
        out_shape=[jax.ShapeDtypeStruct((1, 1), jnp.float32)] * 2,
    )(x)
    return mn[0, 0], mx[0, 0]


def _affine_mm_body(sb_ref, x_ref, w_ref, o_ref):
    xb = x_ref[...] * sb_ref[0] + sb_ref[1]
    o_ref[...] = jnp.dot(xb, w_ref[...], preferred_element_type=jnp.float32)


def _relu_mm_body(x_ref, w_ref, o_ref):
    xb = jnp.maximum(x_ref[...], 0.0)
    o_ref[...] = jnp.dot(xb, w_ref[...], preferred_element_type=jnp.float32)


def _affine_matmul_split(x, w, s, b):
    m, k = x.shape
    n = w.shape[1]
    bm = 2000
    nj = n // 128
    sb = jnp.stack([s, b]).astype(jnp.float32)
    return pl.pallas_call(
        _affine_mm_body,
        grid=(nj, m // bm),
        in_specs=[
            pl.BlockSpec(memory_space=pltpu.SMEM),
            pl.BlockSpec((bm, k), lambda j, i: (i, 0)),
            pl.BlockSpec((k, 128), lambda j, i: (0, j)),
        ],
        out_specs=pl.BlockSpec((bm, 128), lambda j, i: (j * (m // bm) + i, 0)),
        out_shape=jax.ShapeDtypeStruct((nj * m, 128), jnp.float32),
    )(sb, x, w)


def _relu_matmul_split(x, w):
    m, k = x.shape
    n = w.shape[1]
    bm = 2000
    nj = n // 128
    return pl.pallas_call(
        _relu_mm_body,
        grid=(nj, m // bm),
        in_specs=[
            pl.BlockSpec((bm, k), lambda j, i: (i, 0)),
            pl.BlockSpec((k, 128), lambda j, i: (0, j)),
        ],
        out_specs=pl.BlockSpec((bm, 128), lambda j, i: (j * (m // bm) + i, 0)),
        out_shape=jax.ShapeDtypeStruct((nj * m, 128), jnp.float32),
    )(x, w)


_CHUNK = 88
_NCH = 114
_EPT = _NCH * _CHUNK
_NP = 10112
_DRN = 632
_DRNL = 520


def _spmm_sc_kernel(y_hbm, rowr_hbm, colr_hbm, out_hbm,
                    ridx_v, cidx_v, rows0_v, rows1_v, acc_sh, gsem, ssem):
    c = lax.axis_index("c")
    s = lax.axis_index("s")

    pltpu.sync_copy(rowr_hbm.at[s], ridx_v)
    pltpu.sync_copy(colr_hbm.at[c * 16 + s], cidx_v)

    def zfill(i, _):
        for j in range(8):
            rows0_v[i, pl.ds(j * 16, 16)] = jnp.zeros((16,), jnp.float32)
        return 0

    lax.fori_loop(0, _CHUNK, zfill, 0)
    r0 = s * _DRN
    nfull = _DRN // _CHUNK
    for i in range(nfull):
        pltpu.sync_copy(rows0_v, acc_sh.at[pl.ds(r0 + i * _CHUNK, _CHUNK)])
    rem = _DRN - nfull * _CHUNK
    pltpu.sync_copy(rows0_v.at[pl.ds(0, rem)],
                    acc_sh.at[pl.ds(r0 + nfull * _CHUNK, rem)])
    plsc.subcore_barrier()

    def gfire(k, buf):
        pltpu.async_copy(y_hbm.at[cidx_v.at[pl.ds(k * _CHUNK, _CHUNK)]], buf, gsem)

    def gwait(k, buf):
        pltpu.make_async_copy(
            y_hbm.at[cidx_v.at[pl.ds(k * _CHUNK, _CHUNK)]], buf, gsem).wait()

    def sfire(k, buf):
        pltpu.async_copy(buf, acc_sh.at[ridx_v.at[k]], ssem, add=True)

    def swait(k, buf):
        pltpu.make_async_copy(buf, acc_sh.at[ridx_v.at[k]], ssem).wait()

    gfire(0, rows0_v)
    gfire(1, rows1_v)

    def body(j, _):
        k0 = 2 * j
        k1 = k0 + 1
        gwait(k0, rows0_v)
        sfire(k0, rows0_v)
        gwait(k1, rows1_v)
        swait(k0, rows0_v)
        gfire(k0 + 2, rows0_v)
        sfire(k1, rows1_v)
        swait(k1, rows1_v)
        gfire(k1 + 2, rows1_v)
        return 0

    lax.fori_loop(0, _NCH // 2 - 1, body, 0)
    k0 = _NCH - 2
    k1 = _NCH - 1
    gwait(k0, rows0_v)
    sfire(k0, rows0_v)
    gwait(k1, rows1_v)
    swait(k0, rows0_v)
    sfire(k1, rows1_v)
    swait(k1, rows1_v)

    plsc.subcore_barrier()

    @pl.when(s < 15)
    def _():
        pltpu.sync_copy(acc_sh.at[pl.ds(r0, _DRN)],
                        out_hbm.at[pl.ds(c * _N + r0, _DRN)])

    @pl.when(s == 15)
    def _():
        pltpu.sync_copy(acc_sh.at[pl.ds(r0, _DRNL)],
                        out_hbm.at[pl.ds(c * _N + r0, _DRNL)])


def _spmm(ysplit, row, col, base):
    n = _N
    pad = _EPT - (_E // 16)
    row2 = jnp.pad(row.reshape(16, _E // 16), ((0, 0), (0, pad)),
                   constant_values=n)
    col2 = jnp.pad(col.reshape(16, _E // 16), ((0, 0), (0, pad)),
                   constant_values=0) + base
    rowr = row2.reshape(16, _NCH, _CHUNK)
    colcat = jnp.concatenate([col2, col2 + n]).reshape(32, _NCH * _CHUNK)
    ch = 128
    mesh = plsc.VectorSubcoreMesh(core_axis_name="c", subcore_axis_name="s")
    run = pl.kernel(
        _spmm_sc_kernel,
        out_type=jax.ShapeDtypeStruct((2 * n, ch), jnp.float32),
        mesh=mesh,
        scratch_types=[
            pltpu.VMEM((_NCH, _CHUNK), jnp.int32),
            pltpu.VMEM((_NCH * _CHUNK,), jnp.int32),
            pltpu.VMEM((_CHUNK, ch), jnp.float32),
            pltpu.VMEM((_CHUNK, ch), jnp.float32),
            pltpu.VMEM_SHARED((_NP, ch), jnp.float32),
            pltpu.SemaphoreType.DMA,
            pltpu.SemaphoreType.DMA,
        ],
    )
    out = run(ysplit, rowr, colcat)
    return jnp.concatenate([out[:n], out[n:]], axis=1)


def _kld_body(sc_ref, mu_ref, lv_ref, o_ref):
    i = pl.program_id(0)
    mun = mu_ref[...] * sc_ref[0] + sc_ref[1]
    lvn = lv_ref[...] * sc_ref[2] + sc_ref[3]
    e = jnp.exp(lvn)
    t = 1.0 + 2.0 * lvn - mun * mun - e * e
    part = jnp.sum(t)

    @pl.when(i == 0)
    def _():
        o_ref[0, 0] = part

    @pl.when(i > 0)
    def _():
        o_ref[0, 0] = o_ref[0, 0] + part


def _kld_sum(mu, lv, s_mu, b_mu, s_lv, b_lv):
    m, c = mu.shape
    bm = 2000
    sc = jnp.stack([s_mu, b_mu, s_lv, b_lv]).astype(jnp.float32)
    tot = pl.pallas_call(
        _kld_body,
        grid=(m // bm,),
        in_specs=[
            pl.BlockSpec(memory_space=pltpu.SMEM),
            pl.BlockSpec((bm, c), lambda i: (i, 0)),
            pl.BlockSpec((bm, c), lambda i: (i, 0)),
        ],
        out_specs=pl.BlockSpec(memory_space=pltpu.SMEM),
        out_shape=jax.ShapeDtypeStruct((1, 1), jnp.float32),
    )(sc, mu, lv)
    return tot[0, 0]


def _clean(z):
    z = jnp.where(z == -jnp.inf, 0.0, z)
    return jnp.where(jnp.isnan(z), 0.0, z)


def _zzt_body(z1_ref, z2_ref, o_ref):
    a = _clean(z1_ref[...])
    b = _clean(z2_ref[...])
    o_ref[...] = lax.dot_general(
        a, b, (((1,), (1,)), ((), ())), preferred_element_type=jnp.float32
    )


def _zzt(z):
    m, k = z.shape
    bm = 2000
    bn = 2048
    return pl.pallas_call(
        _zzt_body,
        grid=(m // bm, pl.cdiv(m, bn)),
        in_specs=[
            pl.BlockSpec((bm, k), lambda i, j: (i, 0)),
            pl.BlockSpec((bn, k), lambda i, j: (j, 0)),
        ],
        out_specs=pl.BlockSpec((bm, bn), lambda i, j: (i, j)),
        out_shape=jax.ShapeDtypeStruct((m, m), jnp.float32),
    )(z, z)


def kernel(feature, edge_index, W1, W2, W3):
    n = feature.shape[0]
    row = edge_index[0]
    col = edge_index[1]

    mn, mx = _minmax(feature)
    s = 2.0 / (mx - mn)
    b = -2.0 * mn / (mx - mn) - 1.0

    y1s = _affine_matmul_split(feature, W1, s, b)
    s1 = _spmm(y1s, row, col, 0)
    y23s = _relu_matmul_split(s1, jnp.concatenate([W2, W3], axis=1))
    mu = _spmm(y23s, row, col, 0)
    logvar = _spmm(y23s, row, col, 2 * _N)

    adj_recon = _zzt(mu)

    mn_mu, mx_mu, mn_lv, mx_lv = _minmax2(mu, logvar)
    s_mu = 2.0 / (mx_mu - mn_mu)
    b_mu = -2.0 * mn_mu / (mx_mu - mn_mu) - 1.0
    s_lv = 2.0 / (mx_lv - mn_lv)
    b_lv = -2.0 * mn_lv / (mx_lv - mn_lv) - 1.0
    tot = _kld_sum(mu, logvar, s_mu, b_mu, s_lv, b_lv)
    kld = (-0.5 / n) * (tot / n)

    return (adj_recon, mu, logvar, mu, kld)
